# baseline (speedup 1.0000x reference)
import functools

import jax
import jax.numpy as jnp
from jax import lax
from jax.experimental import pallas as pl
from jax.experimental.pallas import tpu as pltpu
from jax.experimental.pallas import tpu_sc as plsc

N = 10000
NP = 10240
E = 320000
EP = 327680
NTILES = 32
EPT = EP // NTILES
K = 128
CHUNKS = EPT // K
CH0 = 74
CH1 = 86
RPT = NP // 16
D1 = 64
D2 = 48
H1 = 8

_f32 = jnp.float32


def _dense1_body(x_ref, w_ref, ss_ref, sd_ref, h_ref, as_ref, ad_ref, es_ref):
  h = jnp.dot(x_ref[...], w_ref[...], preferred_element_type=_f32)
  h_ref[...] = h
  a_s = jnp.dot(h, ss_ref[...], preferred_element_type=_f32)
  a_d = jnp.dot(h, sd_ref[...], preferred_element_type=_f32)
  as_ref[...] = a_s
  ad_ref[...] = a_d
  z = a_s + a_d
  es_ref[...] = jnp.exp(jnp.maximum(z, 0.2 * z))


def _dense2_body(q_ref, d_ref, h1_ref, es1_ref, rm_ref, b1_ref, w2_ref,
                 a2_ref, h2_ref, as2_ref, ad2_ref, es2_ref):
  q = q_ref[0, :N, :] + q_ref[1, :N, :]
  h1 = h1_ref[...]
  es1 = es1_ref[...]
  rm = rm_ref[...]
  den = d_ref[0, :N, :] + d_ref[1, :N, :] + es1 + 1e-16
  recip = 1.0 / den
  num = q + jnp.dot(es1, rm, preferred_element_type=_f32) * h1
  out1 = num * jnp.dot(recip, rm, preferred_element_type=_f32)
  x2 = out1 + b1_ref[...]
  x2 = jnp.where(x2 > 0, x2, jnp.exp(jnp.minimum(x2, 0.0)) - 1.0)
  h2 = jnp.dot(x2, w2_ref[...], preferred_element_type=_f32)
  h2_ref[...] = h2
  a2 = jnp.dot(h2, a2_ref[...], preferred_element_type=_f32)
  a_s = a2[:, 0:1]
  a_d = a2[:, 1:2]
  as2_ref[...] = a_s
  ad2_ref[...] = a_d
  z = a_s + a_d
  es2_ref[...] = jnp.exp(jnp.maximum(z, 0.2 * z))


def _final_body(q_ref, h2_ref, es2_ref, b2_ref, o_ref):
  q = q_ref[0, :N, :] + q_ref[1, :N, :]
  es2 = es2_ref[...]
  num = q[:, :40] + es2 * h2_ref[...]
  den = q[:, 40:41] + es2 + 1e-16
  o = num / den + b2_ref[...]
  m = jnp.max(o, axis=-1, keepdims=True)
  o = o - m
  o_ref[...] = o - jnp.log(jnp.sum(jnp.exp(o), axis=-1, keepdims=True))


def _edge1_body(src_h, dst_h, t_h, as_h, ad_h, out_h, den_h,
                ACC, DEN, sv0, sv1, dv0, dv1, ga0, ga1, gd0, gd1, tr0, tr1,
                msg0, msg1, ed0, ed1, ee,
                sa0, sa1, sd0, sd1, st0, st1, sc0, sc1, se0, se1):
  c = lax.axis_index("c")
  s = lax.axis_index("s")
  r0 = s * RPT
  SL = ((sv0, dv0, ga0, gd0, tr0, msg0, ed0, sa0, sd0, st0, sc0, se0),
        (sv1, dv1, ga1, gd1, tr1, msg1, ed1, sa1, sd1, st1, sc1, se1))
  i16 = lax.iota(jnp.int32, 16)
  half = lax.shift_right_logical(i16, jnp.full((16,), 3, jnp.int32))
  three = jnp.full((16,), 3, jnp.int32)
  seven = jnp.full((16,), 7, jnp.int32)
  p2 = jnp.full((16,), 0.2, _f32)
  chc = jnp.where(c == 0, CH0, CH1)
  ebase = c * (16 * CH0 * K) + s * (chc * K)
  zeros16 = jnp.zeros((16,), _f32)

  def fire(slot, k):
    sv, dv, ga, gd, tr, _, _, sa, sd, st, _, _ = SL[slot]
    base = ebase + k * K
    pltpu.sync_copy(src_h.at[pl.ds(base, K)], sv)
    pltpu.sync_copy(dst_h.at[pl.ds(base, K)], dv)
    pltpu.async_copy(as_h.at[sv], ga, sa)
    pltpu.async_copy(ad_h.at[dv], gd, sd)
    pltpu.async_copy(t_h.at[sv], tr, st)

  def wait_gathers(slot):
    sv, dv, ga, gd, tr, _, _, sa, sd, st, _, _ = SL[slot]
    pltpu.make_async_copy(as_h.at[sv], ga, sa).wait()
    pltpu.make_async_copy(ad_h.at[dv], gd, sd).wait()
    pltpu.make_async_copy(t_h.at[sv], tr, st).wait()

  def compute(slot):
    _, _, ga, gd, tr, msg, ed, _, _, _, _, _ = SL[slot]

    def _e(j, _):
      flat = jnp.full((16,), j * 16, jnp.int32) + i16
      r = lax.shift_right_logical(flat, three)
      col = lax.bitwise_and(flat, seven)
      z = plsc.load_gather(ga, [r, col]) + plsc.load_gather(gd, [r, col])
      ev = jnp.exp(jnp.maximum(z, z * p2))
      ee[pl.ds(j * 16, 16)] = ev
      plsc.store_scatter(ed, [r, col], ev)
      return None
    lax.fori_loop(0, K * H1 // 16, _e, None)

    def _m(e, _):
      eb = e * H1
      for q in range(4):
        mult = plsc.load_gather(
            ee, [jnp.full((16,), eb + 2 * q, jnp.int32) + half])
        msg[e, pl.ds(q * 16, 16)] = tr[e, pl.ds(q * 16, 16)] * mult
      return None
    lax.fori_loop(0, K, _m, None)

  def fire_scatter(slot):
    _, dv, _, _, _, msg, ed, _, _, _, sc, se = SL[slot]
    pltpu.async_copy(msg, ACC.at[dv], sc, add=True)
    pltpu.async_copy(ed, DEN.at[dv], se, add=True)

  def wait_scatter(slot):
    _, dv, _, _, _, msg, ed, _, _, _, sc, se = SL[slot]
    pltpu.make_async_copy(msg, ACC.at[dv], sc).wait()
    pltpu.make_async_copy(ed, DEN.at[dv], se).wait()

  fire(0, 0)

  def _z(i, _):
    for q in range(D1 // 16):
      msg1[i, pl.ds(q * 16, 16)] = zeros16
    return None
  lax.fori_loop(0, K, _z, None)

  def _ze(j, _):
    flat = jnp.full((16,), j * 16, jnp.int32) + i16
    r = lax.shift_right_logical(flat, three)
    col = lax.bitwise_and(flat, seven)
    plsc.store_scatter(ed1, [r, col], zeros16)
    return None
  lax.fori_loop(0, K * H1 // 16, _ze, None)

  def _zc(j, _):
    pltpu.sync_copy(msg1, ACC.at[pl.ds(r0 + j * K, K)])
    pltpu.sync_copy(ed1, DEN.at[pl.ds(r0 + j * K, K)])
    return None
  lax.fori_loop(0, RPT // K, _zc, None)
  plsc.subcore_barrier()

  def body(i, _):
    @pl.when(i > 0)
    def _():
      wait_scatter(1)
    fire(1, 2 * i + 1)
    wait_gathers(0)
    compute(0)
    fire_scatter(0)
    wait_gathers(1)
    compute(1)
    wait_scatter(0)

    @pl.when(2 * i + 2 < chc)
    def _():
      fire(0, 2 * i + 2)
    fire_scatter(1)
    return None
  lax.fori_loop(0, chc // 2, body, None)
  wait_scatter(1)

  plsc.subcore_barrier()
  pltpu.sync_copy(ACC.at[pl.ds(r0, RPT)], out_h.at[c, pl.ds(r0, RPT)])
  pltpu.sync_copy(DEN.at[pl.ds(r0, RPT)], den_h.at[c, pl.ds(r0, RPT)])


def _edge2_body(src_h, dst_h, t_h, as_h, ad_h, out_h,
                ACC, AS, AD, sv0, sv1, dv0, dv1, tr0, tr1, msg0, msg1, ee,
                st0, st1, sc0, sc1):
  c = lax.axis_index("c")
  s = lax.axis_index("s")
  r0 = s * RPT
  SL = ((sv0, dv0, tr0, msg0, st0, sc0),
        (sv1, dv1, tr1, msg1, st1, sc1))
  chc = jnp.where(c == 0, CH0, CH1)
  ebase = c * (16 * CH0 * K) + s * (chc * K)
  zeros16 = jnp.zeros((16,), _f32)
  p2 = jnp.full((16,), 0.2, _f32)

  def fire(slot, k):
    sv, dv, tr, _, st, _ = SL[slot]
    base = ebase + k * K
    pltpu.sync_copy(src_h.at[pl.ds(base, K)], sv)
    pltpu.sync_copy(dst_h.at[pl.ds(base, K)], dv)
    pltpu.async_copy(t_h.at[sv], tr, st)

  def wait_gathers(slot):
    sv, _, tr, _, st, _ = SL[slot]
    pltpu.make_async_copy(t_h.at[sv], tr, st).wait()

  def compute(slot):
    sv, dv, tr, msg, _, _ = SL[slot]

    def _e(j, _):
      svv = sv[pl.ds(j * 16, 16)]
      dvv = dv[pl.ds(j * 16, 16)]
      z = plsc.load_gather(AS, [svv]) + plsc.load_gather(AD, [dvv])
      ee[pl.ds(j * 16, 16)] = jnp.exp(jnp.maximum(z, z * p2))
      return None
    lax.fori_loop(0, K // 16, _e, None)

    def _m(t, _):
      tb = t * 16
      for e0 in range(16):
        mult = plsc.load_gather(ee, [jnp.full((16,), tb + e0, jnp.int32)])
        erow = tb + e0
        for q in range(3):
          msg[erow, pl.ds(q * 16, 16)] = tr[erow, pl.ds(q * 16, 16)] * mult
      return None
    lax.fori_loop(0, K // 16, _m, None)

  def fire_scatter(slot):
    _, dv, _, msg, _, sc = SL[slot]
    pltpu.async_copy(msg, ACC.at[dv], sc, add=True)

  def wait_scatter(slot):
    _, dv, _, msg, _, sc = SL[slot]
    pltpu.make_async_copy(msg, ACC.at[dv], sc).wait()

  fire(0, 0)
  pltpu.sync_copy(as_h, AS)
  pltpu.sync_copy(ad_h, AD)

  def _z(i, _):
    for q in range(D2 // 16):
      msg1[i, pl.ds(q * 16, 16)] = zeros16
    return None
  lax.fori_loop(0, K, _z, None)

  def _zc(j, _):
    pltpu.sync_copy(msg1, ACC.at[pl.ds(r0 + j * K, K)])
    return None
  lax.fori_loop(0, RPT // K, _zc, None)
  plsc.subcore_barrier()

  def body(i, _):
    @pl.when(i > 0)
    def _():
      wait_scatter(1)
    fire(1, 2 * i + 1)
    wait_gathers(0)
    compute(0)
    fire_scatter(0)
    wait_gathers(1)
    compute(1)
    wait_scatter(0)

    @pl.when(2 * i + 2 < chc)
    def _():
      fire(0, 2 * i + 2)
    fire_scatter(1)
    return None
  lax.fori_loop(0, chc // 2, body, None)
  wait_scatter(1)

  plsc.subcore_barrier()
  pltpu.sync_copy(ACC.at[pl.ds(r0, RPT)], out_h.at[c, pl.ds(r0, RPT)])


_sc_mesh = plsc.VectorSubcoreMesh(core_axis_name="c", subcore_axis_name="s")

_sc_params = pltpu.CompilerParams(
    needs_layout_passes=False, use_tc_tiling_on_sc=False)

_edge1 = functools.partial(
    pl.kernel,
    _edge1_body,
    out_type=[
        jax.ShapeDtypeStruct((2, NP, D1), _f32),
        jax.ShapeDtypeStruct((2, NP, H1), _f32),
    ],
    mesh=_sc_mesh,
    compiler_params=_sc_params,
    scratch_types=(
        [pltpu.VMEM_SHARED((NP, D1), _f32)]
        + [pltpu.VMEM_SHARED((NP, H1), _f32)]
        + [pltpu.VMEM((K,), jnp.int32)] * 4
        + [pltpu.VMEM((K, H1), _f32)] * 4
        + [pltpu.VMEM((K, D1), _f32)] * 4
        + [pltpu.VMEM((K, H1), _f32)] * 2
        + [pltpu.VMEM((K * H1,), _f32)]
        + [pltpu.SemaphoreType.DMA] * 10
    ),
)

_edge2 = functools.partial(
    pl.kernel,
    _edge2_body,
    out_type=jax.ShapeDtypeStruct((2, NP, D2), _f32),
    mesh=_sc_mesh,
    compiler_params=_sc_params,
    scratch_types=(
        [pltpu.VMEM_SHARED((NP, D2), _f32)]
        + [pltpu.VMEM((NP,), _f32)] * 2
        + [pltpu.VMEM((K,), jnp.int32)] * 4
        + [pltpu.VMEM((K, D2), _f32)] * 4
        + [pltpu.VMEM((K,), _f32)]
        + [pltpu.SemaphoreType.DMA] * 4
    ),
)


def kernel(x, edge_index, W1, att_src1, att_dst1, b1, W2, att_src2, att_dst2,
           b2):
  a_s1 = att_src1.reshape(H1, H1)
  a_d1 = att_dst1.reshape(H1, H1)
  eye = jnp.eye(H1, dtype=_f32)
  S_src = (a_s1[:, :, None] * eye[:, None, :]).reshape(H1 * H1, H1)
  S_dst = (a_d1[:, :, None] * eye[:, None, :]).reshape(H1 * H1, H1)
  RM = jnp.repeat(eye, H1, axis=1)
  A2 = jnp.stack([att_src2.reshape(40), att_dst2.reshape(40)], axis=1)

  pad = jnp.full((EP - E,), N, jnp.int32)
  src = jnp.concatenate([edge_index[0], pad])
  dst = jnp.concatenate([edge_index[1], pad])

  h1, as1, ad1, es1 = pl.pallas_call(
      _dense1_body,
      out_shape=[
          jax.ShapeDtypeStruct((N, 64), _f32),
          jax.ShapeDtypeStruct((N, H1), _f32),
          jax.ShapeDtypeStruct((N, H1), _f32),
          jax.ShapeDtypeStruct((N, H1), _f32),
      ],
  )(x, W1, S_src, S_dst)

  T1 = jnp.zeros((NP, D1), _f32).at[:N].set(h1)
  as1p = jnp.zeros((NP, H1), _f32).at[:N].set(as1)
  ad1p = jnp.zeros((NP, H1), _f32).at[:N].set(ad1)
  q1, d1 = _edge1()(src, dst, T1, as1p, ad1p)

  h2, as2, ad2, es2 = pl.pallas_call(
      _dense2_body,
      out_shape=[
          jax.ShapeDtypeStruct((N, 40), _f32),
          jax.ShapeDtypeStruct((N, 1), _f32),
          jax.ShapeDtypeStruct((N, 1), _f32),
          jax.ShapeDtypeStruct((N, 1), _f32),
      ],
  )(q1, d1, h1, es1, RM, b1, W2, A2)

  T2 = jnp.zeros((NP, D2), _f32)
  T2 = T2.at[:N, :40].set(h2).at[:, 40].set(1.0)
  as2p = jnp.zeros((NP,), _f32).at[:N].set(as2[:, 0])
  ad2p = jnp.zeros((NP,), _f32).at[:N].set(ad2[:, 0])
  q2 = _edge2()(src, dst, T2, as2p, ad2p)

  out = pl.pallas_call(
      _final_body,
      out_shape=jax.ShapeDtypeStruct((N, 40), _f32),
  )(q2, h2, es2, b2)
  return out


# speedup vs baseline: 1.0873x; 1.0873x over previous
import functools

import jax
import jax.numpy as jnp
from jax import lax
from jax.experimental import pallas as pl
from jax.experimental.pallas import tpu as pltpu
from jax.experimental.pallas import tpu_sc as plsc

N = 10000
NP = 10240
E = 320000
EP = 327680
NTILES = 32
EPT = EP // NTILES
K = 128
CHUNKS = EPT // K
CH0 = 86
CH1 = 74
RPT = NP // 16
D1 = 64
D2 = 48
H1 = 8

_f32 = jnp.float32


def _dense1_body(x_ref, w_ref, ss_ref, sd_ref, h_ref, as_ref, ad_ref, es_ref):
  h = jnp.dot(x_ref[...], w_ref[...], preferred_element_type=_f32)
  h_ref[...] = h
  a_s = jnp.dot(h, ss_ref[...], preferred_element_type=_f32)
  a_d = jnp.dot(h, sd_ref[...], preferred_element_type=_f32)
  as_ref[...] = a_s
  ad_ref[...] = a_d
  z = a_s + a_d
  es_ref[...] = jnp.exp(jnp.maximum(z, 0.2 * z))


def _dense2_body(q_ref, d_ref, h1_ref, es1_ref, rm_ref, b1_ref, w2_ref,
                 a2_ref, h2_ref, as2_ref, ad2_ref, es2_ref):
  q = q_ref[0, :N, :] + q_ref[1, :N, :]
  h1 = h1_ref[...]
  es1 = es1_ref[...]
  rm = rm_ref[...]
  den = d_ref[0, :N, :] + d_ref[1, :N, :] + es1 + 1e-16
  recip = 1.0 / den
  num = q + jnp.dot(es1, rm, preferred_element_type=_f32) * h1
  out1 = num * jnp.dot(recip, rm, preferred_element_type=_f32)
  x2 = out1 + b1_ref[...]
  x2 = jnp.where(x2 > 0, x2, jnp.exp(jnp.minimum(x2, 0.0)) - 1.0)
  h2 = jnp.dot(x2, w2_ref[...], preferred_element_type=_f32)
  h2_ref[...] = h2
  a2 = jnp.dot(h2, a2_ref[...], preferred_element_type=_f32)
  a_s = a2[:, 0:1]
  a_d = a2[:, 1:2]
  as2_ref[...] = a_s
  ad2_ref[...] = a_d
  z = a_s + a_d
  es2_ref[...] = jnp.exp(jnp.maximum(z, 0.2 * z))


def _final_body(q_ref, h2_ref, es2_ref, b2_ref, o_ref):
  q = q_ref[0, :N, :] + q_ref[1, :N, :]
  es2 = es2_ref[...]
  num = q[:, :40] + es2 * h2_ref[...]
  den = q[:, 40:41] + es2 + 1e-16
  o = num / den + b2_ref[...]
  m = jnp.max(o, axis=-1, keepdims=True)
  o = o - m
  o_ref[...] = o - jnp.log(jnp.sum(jnp.exp(o), axis=-1, keepdims=True))


def _edge1_body(src_h, dst_h, t_h, as_h, ad_h, out_h, den_h,
                ACC, DEN, sv0, sv1, dv0, dv1, ga0, ga1, gd0, gd1, tr0, tr1,
                msg0, msg1, ed0, ed1, ee,
                sa0, sa1, sd0, sd1, st0, st1, sc0, sc1, se0, se1):
  c = lax.axis_index("c")
  s = lax.axis_index("s")
  r0 = s * RPT
  SL = ((sv0, dv0, ga0, gd0, tr0, msg0, ed0, sa0, sd0, st0, sc0, se0),
        (sv1, dv1, ga1, gd1, tr1, msg1, ed1, sa1, sd1, st1, sc1, se1))
  i16 = lax.iota(jnp.int32, 16)
  half = lax.shift_right_logical(i16, jnp.full((16,), 3, jnp.int32))
  three = jnp.full((16,), 3, jnp.int32)
  seven = jnp.full((16,), 7, jnp.int32)
  p2 = jnp.full((16,), 0.2, _f32)
  chc = jnp.where(c == 0, CH0, CH1)
  ebase = c * (16 * CH0 * K) + s * (chc * K)
  zeros16 = jnp.zeros((16,), _f32)

  def fire(slot, k):
    sv, dv, ga, gd, tr, _, _, sa, sd, st, _, _ = SL[slot]
    base = ebase + k * K
    pltpu.sync_copy(src_h.at[pl.ds(base, K)], sv)
    pltpu.sync_copy(dst_h.at[pl.ds(base, K)], dv)
    pltpu.async_copy(as_h.at[sv], ga, sa)
    pltpu.async_copy(ad_h.at[dv], gd, sd)
    pltpu.async_copy(t_h.at[sv], tr, st)

  def wait_gathers(slot):
    sv, dv, ga, gd, tr, _, _, sa, sd, st, _, _ = SL[slot]
    pltpu.make_async_copy(as_h.at[sv], ga, sa).wait()
    pltpu.make_async_copy(ad_h.at[dv], gd, sd).wait()
    pltpu.make_async_copy(t_h.at[sv], tr, st).wait()

  def compute(slot):
    _, _, ga, gd, tr, msg, ed, _, _, _, _, _ = SL[slot]

    def _e(j, _):
      flat = jnp.full((16,), j * 16, jnp.int32) + i16
      r = lax.shift_right_logical(flat, three)
      col = lax.bitwise_and(flat, seven)
      z = plsc.load_gather(ga, [r, col]) + plsc.load_gather(gd, [r, col])
      ev = jnp.exp(jnp.maximum(z, z * p2))
      ee[pl.ds(j * 16, 16)] = ev
      plsc.store_scatter(ed, [r, col], ev)
      return None
    lax.fori_loop(0, K * H1 // 16, _e, None)

    def _m(e, _):
      eb = e * H1
      for q in range(4):
        mult = plsc.load_gather(
            ee, [jnp.full((16,), eb + 2 * q, jnp.int32) + half])
        msg[e, pl.ds(q * 16, 16)] = tr[e, pl.ds(q * 16, 16)] * mult
      return None
    lax.fori_loop(0, K, _m, None)

  def fire_scatter(slot):
    _, dv, _, _, _, msg, ed, _, _, _, sc, se = SL[slot]
    pltpu.async_copy(msg, ACC.at[dv], sc, add=True)
    pltpu.async_copy(ed, DEN.at[dv], se, add=True)

  def wait_scatter(slot):
    _, dv, _, _, _, msg, ed, _, _, _, sc, se = SL[slot]
    pltpu.make_async_copy(msg, ACC.at[dv], sc).wait()
    pltpu.make_async_copy(ed, DEN.at[dv], se).wait()

  fire(0, 0)

  def _z(i, _):
    for q in range(D1 // 16):
      msg1[i, pl.ds(q * 16, 16)] = zeros16
    return None
  lax.fori_loop(0, K, _z, None)

  def _ze(j, _):
    flat = jnp.full((16,), j * 16, jnp.int32) + i16
    r = lax.shift_right_logical(flat, three)
    col = lax.bitwise_and(flat, seven)
    plsc.store_scatter(ed1, [r, col], zeros16)
    return None
  lax.fori_loop(0, K * H1 // 16, _ze, None)

  def _zc(j, _):
    pltpu.sync_copy(msg1, ACC.at[pl.ds(r0 + j * K, K)])
    pltpu.sync_copy(ed1, DEN.at[pl.ds(r0 + j * K, K)])
    return None
  lax.fori_loop(0, RPT // K, _zc, None)
  plsc.subcore_barrier()

  def body(i, _):
    @pl.when(i > 0)
    def _():
      wait_scatter(1)
    fire(1, 2 * i + 1)
    wait_gathers(0)
    compute(0)
    fire_scatter(0)
    wait_gathers(1)
    compute(1)
    wait_scatter(0)

    @pl.when(2 * i + 2 < chc)
    def _():
      fire(0, 2 * i + 2)
    fire_scatter(1)
    return None
  lax.fori_loop(0, chc // 2, body, None)
  wait_scatter(1)

  plsc.subcore_barrier()
  pltpu.sync_copy(ACC.at[pl.ds(r0, RPT)], out_h.at[c, pl.ds(r0, RPT)])
  pltpu.sync_copy(DEN.at[pl.ds(r0, RPT)], den_h.at[c, pl.ds(r0, RPT)])


def _edge2_body(src_h, dst_h, t_h, as_h, ad_h, out_h,
                ACC, AS, AD, sv0, sv1, dv0, dv1, tr0, tr1, msg0, msg1, ee,
                st0, st1, sc0, sc1):
  c = lax.axis_index("c")
  s = lax.axis_index("s")
  r0 = s * RPT
  SL = ((sv0, dv0, tr0, msg0, st0, sc0),
        (sv1, dv1, tr1, msg1, st1, sc1))
  chc = jnp.where(c == 0, CH0, CH1)
  ebase = c * (16 * CH0 * K) + s * (chc * K)
  zeros16 = jnp.zeros((16,), _f32)
  p2 = jnp.full((16,), 0.2, _f32)

  def fire(slot, k):
    sv, dv, tr, _, st, _ = SL[slot]
    base = ebase + k * K
    pltpu.sync_copy(src_h.at[pl.ds(base, K)], sv)
    pltpu.sync_copy(dst_h.at[pl.ds(base, K)], dv)
    pltpu.async_copy(t_h.at[sv], tr, st)

  def wait_gathers(slot):
    sv, _, tr, _, st, _ = SL[slot]
    pltpu.make_async_copy(t_h.at[sv], tr, st).wait()

  def compute(slot):
    sv, dv, tr, msg, _, _ = SL[slot]

    def _e(j, _):
      svv = sv[pl.ds(j * 16, 16)]
      dvv = dv[pl.ds(j * 16, 16)]
      z = plsc.load_gather(AS, [svv]) + plsc.load_gather(AD, [dvv])
      ee[pl.ds(j * 16, 16)] = jnp.exp(jnp.maximum(z, z * p2))
      return None
    lax.fori_loop(0, K // 16, _e, None)

    def _m(t, _):
      tb = t * 16
      for e0 in range(16):
        mult = plsc.load_gather(ee, [jnp.full((16,), tb + e0, jnp.int32)])
        erow = tb + e0
        for q in range(3):
          msg[erow, pl.ds(q * 16, 16)] = tr[erow, pl.ds(q * 16, 16)] * mult
      return None
    lax.fori_loop(0, K // 16, _m, None)

  def fire_scatter(slot):
    _, dv, _, msg, _, sc = SL[slot]
    pltpu.async_copy(msg, ACC.at[dv], sc, add=True)

  def wait_scatter(slot):
    _, dv, _, msg, _, sc = SL[slot]
    pltpu.make_async_copy(msg, ACC.at[dv], sc).wait()

  fire(0, 0)
  pltpu.sync_copy(as_h, AS)
  pltpu.sync_copy(ad_h, AD)

  def _z(i, _):
    for q in range(D2 // 16):
      msg1[i, pl.ds(q * 16, 16)] = zeros16
    return None
  lax.fori_loop(0, K, _z, None)

  def _zc(j, _):
    pltpu.sync_copy(msg1, ACC.at[pl.ds(r0 + j * K, K)])
    return None
  lax.fori_loop(0, RPT // K, _zc, None)
  plsc.subcore_barrier()

  def body(i, _):
    @pl.when(i > 0)
    def _():
      wait_scatter(1)
    fire(1, 2 * i + 1)
    wait_gathers(0)
    compute(0)
    fire_scatter(0)
    wait_gathers(1)
    compute(1)
    wait_scatter(0)

    @pl.when(2 * i + 2 < chc)
    def _():
      fire(0, 2 * i + 2)
    fire_scatter(1)
    return None
  lax.fori_loop(0, chc // 2, body, None)
  wait_scatter(1)

  plsc.subcore_barrier()
  pltpu.sync_copy(ACC.at[pl.ds(r0, RPT)], out_h.at[c, pl.ds(r0, RPT)])


_sc_mesh = plsc.VectorSubcoreMesh(core_axis_name="c", subcore_axis_name="s")

_sc_params = pltpu.CompilerParams(
    needs_layout_passes=False, use_tc_tiling_on_sc=False)

_edge1 = functools.partial(
    pl.kernel,
    _edge1_body,
    out_type=[
        jax.ShapeDtypeStruct((2, NP, D1), _f32),
        jax.ShapeDtypeStruct((2, NP, H1), _f32),
    ],
    mesh=_sc_mesh,
    compiler_params=_sc_params,
    scratch_types=(
        [pltpu.VMEM_SHARED((NP, D1), _f32)]
        + [pltpu.VMEM_SHARED((NP, H1), _f32)]
        + [pltpu.VMEM((K,), jnp.int32)] * 4
        + [pltpu.VMEM((K, H1), _f32)] * 4
        + [pltpu.VMEM((K, D1), _f32)] * 4
        + [pltpu.VMEM((K, H1), _f32)] * 2
        + [pltpu.VMEM((K * H1,), _f32)]
        + [pltpu.SemaphoreType.DMA] * 10
    ),
)

_edge2 = functools.partial(
    pl.kernel,
    _edge2_body,
    out_type=jax.ShapeDtypeStruct((2, NP, D2), _f32),
    mesh=_sc_mesh,
    compiler_params=_sc_params,
    scratch_types=(
        [pltpu.VMEM_SHARED((NP, D2), _f32)]
        + [pltpu.VMEM((NP,), _f32)] * 2
        + [pltpu.VMEM((K,), jnp.int32)] * 4
        + [pltpu.VMEM((K, D2), _f32)] * 4
        + [pltpu.VMEM((K,), _f32)]
        + [pltpu.SemaphoreType.DMA] * 4
    ),
)


def kernel(x, edge_index, W1, att_src1, att_dst1, b1, W2, att_src2, att_dst2,
           b2):
  a_s1 = att_src1.reshape(H1, H1)
  a_d1 = att_dst1.reshape(H1, H1)
  eye = jnp.eye(H1, dtype=_f32)
  S_src = (a_s1[:, :, None] * eye[:, None, :]).reshape(H1 * H1, H1)
  S_dst = (a_d1[:, :, None] * eye[:, None, :]).reshape(H1 * H1, H1)
  RM = jnp.repeat(eye, H1, axis=1)
  A2 = jnp.stack([att_src2.reshape(40), att_dst2.reshape(40)], axis=1)

  pad = jnp.full((EP - E,), N, jnp.int32)
  src = jnp.concatenate([edge_index[0], pad])
  dst = jnp.concatenate([edge_index[1], pad])

  h1, as1, ad1, es1 = pl.pallas_call(
      _dense1_body,
      out_shape=[
          jax.ShapeDtypeStruct((N, 64), _f32),
          jax.ShapeDtypeStruct((N, H1), _f32),
          jax.ShapeDtypeStruct((N, H1), _f32),
          jax.ShapeDtypeStruct((N, H1), _f32),
      ],
  )(x, W1, S_src, S_dst)

  T1 = jnp.zeros((NP, D1), _f32).at[:N].set(h1)
  as1p = jnp.zeros((NP, H1), _f32).at[:N].set(as1)
  ad1p = jnp.zeros((NP, H1), _f32).at[:N].set(ad1)
  q1, d1 = _edge1()(src, dst, T1, as1p, ad1p)

  h2, as2, ad2, es2 = pl.pallas_call(
      _dense2_body,
      out_shape=[
          jax.ShapeDtypeStruct((N, 40), _f32),
          jax.ShapeDtypeStruct((N, 1), _f32),
          jax.ShapeDtypeStruct((N, 1), _f32),
          jax.ShapeDtypeStruct((N, 1), _f32),
      ],
  )(q1, d1, h1, es1, RM, b1, W2, A2)

  T2 = jnp.zeros((NP, D2), _f32)
  T2 = T2.at[:N, :40].set(h2).at[:, 40].set(1.0)
  as2p = jnp.zeros((NP,), _f32).at[:N].set(as2[:, 0])
  ad2p = jnp.zeros((NP,), _f32).at[:N].set(ad2[:, 0])
  q2 = _edge2()(src, dst, T2, as2p, ad2p)

  out = pl.pallas_call(
      _final_body,
      out_shape=jax.ShapeDtypeStruct((N, 40), _f32),
  )(q2, h2, es2, b2)
  return out
